# Initial kernel scaffold; baseline (speedup 1.0000x reference)
import functools
import jax
import jax.numpy as jnp
from jax import lax
from jax.experimental import pallas as pl
from jax.experimental.pallas import tpu as pltpu
from jax.experimental.pallas import tpu_sc as plsc

N_NODES = 10000
N_EDGES = 320000
NODE_DIM = 128
EDGE_EMB = 16
NUM_GRAPHS = 256

NC = 2
NS = 16
NW = NC * NS
CH = 80
PER_TILE = N_EDGES // NW
N_CHUNK = PER_TILE // CH
ROWS_PER_TILE = N_NODES // NS
ZCH = 125

_mesh = plsc.VectorSubcoreMesh(
    core_axis_name="c", subcore_axis_name="s", num_cores=NC, num_subcores=NS)


@functools.partial(
    pl.kernel, mesh=_mesh,
    out_type=jax.ShapeDtypeStruct((NC * N_NODES, NODE_DIM), jnp.float32),
    scratch_types=[
        pltpu.VMEM((CH,), jnp.int32),
        pltpu.VMEM((CH,), jnp.int32),
        pltpu.VMEM((CH, NODE_DIM), jnp.float32),
        pltpu.VMEM((ZCH, NODE_DIM), jnp.float32),
        pltpu.VMEM_SHARED((N_NODES, NODE_DIM), jnp.float32),
        pltpu.SemaphoreType.DMA,
    ])
def _seg_sum_sc(h_hbm, src_hbm, dst_hbm, out_hbm, src_v, dst_v, rows_v,
                zbuf, acc_sh, sem):
    c = lax.axis_index("c")
    s = lax.axis_index("s")
    wid = c * NS + s
    zero16 = jnp.zeros((16,), jnp.float32)

    def zrow(i, carry):
        for j in range(8):
            zbuf[i, pl.ds(j * 16, 16)] = zero16
        return carry
    lax.fori_loop(0, ZCH, zrow, 0)

    base_r = s * ROWS_PER_TILE
    for k in range(ROWS_PER_TILE // ZCH):
        pltpu.sync_copy(zbuf, acc_sh.at[pl.ds(base_r + k * ZCH, ZCH)])
    plsc.subcore_barrier()

    ebase = wid * PER_TILE

    def body(i, carry):
        off = ebase + i * CH
        pltpu.sync_copy(src_hbm.at[pl.ds(off, CH)], src_v)
        pltpu.sync_copy(dst_hbm.at[pl.ds(off, CH)], dst_v)
        pltpu.async_copy(h_hbm.at[src_v], rows_v, sem).wait()
        pltpu.sync_copy(rows_v, acc_sh.at[dst_v], add=True)
        return carry
    lax.fori_loop(0, N_CHUNK, body, 0)
    plsc.subcore_barrier()

    out_base = c * N_NODES + base_r
    for k in range(ROWS_PER_TILE // ZCH):
        pltpu.sync_copy(acc_sh.at[pl.ds(base_r + k * ZCH, ZCH)], zbuf)
        pltpu.sync_copy(zbuf, out_hbm.at[pl.ds(out_base + k * ZCH, ZCH)])


@functools.partial(
    pl.kernel, mesh=_mesh,
    out_type=(jax.ShapeDtypeStruct((NC * N_NODES, EDGE_EMB), jnp.float32),
              jax.ShapeDtypeStruct((NC * N_NODES, EDGE_EMB), jnp.float32)),
    scratch_types=[
        pltpu.VMEM((CH,), jnp.int32),
        pltpu.VMEM((CH, EDGE_EMB), jnp.float32),
        pltpu.VMEM((CH, EDGE_EMB), jnp.float32),
        pltpu.VMEM((ZCH, EDGE_EMB), jnp.float32),
        pltpu.VMEM_SHARED((N_NODES, EDGE_EMB), jnp.float32),
        pltpu.VMEM_SHARED((N_NODES, EDGE_EMB), jnp.float32),
        pltpu.SemaphoreType.DMA,
    ])
def _edge_agg_sc(ef_hbm, dst_hbm, efout_hbm, cntout_hbm, dst_v, rows_v,
                 ones_v, zbuf, efacc_sh, cntacc_sh, sem):
    c = lax.axis_index("c")
    s = lax.axis_index("s")
    wid = c * NS + s
    zero16 = jnp.zeros((16,), jnp.float32)
    one16 = jnp.ones((16,), jnp.float32)

    def zrow(i, carry):
        zbuf[i] = zero16
        return carry
    lax.fori_loop(0, ZCH, zrow, 0)

    def orow(i, carry):
        ones_v[i] = one16
        return carry
    lax.fori_loop(0, CH, orow, 0)

    base_r = s * ROWS_PER_TILE
    for k in range(ROWS_PER_TILE // ZCH):
        pltpu.sync_copy(zbuf, efacc_sh.at[pl.ds(base_r + k * ZCH, ZCH)])
        pltpu.sync_copy(zbuf, cntacc_sh.at[pl.ds(base_r + k * ZCH, ZCH)])
    plsc.subcore_barrier()

    ebase = wid * PER_TILE

    def body(i, carry):
        off = ebase + i * CH
        pltpu.sync_copy(dst_hbm.at[pl.ds(off, CH)], dst_v)
        pltpu.sync_copy(ef_hbm.at[pl.ds(off, CH)], rows_v)
        pltpu.sync_copy(rows_v, efacc_sh.at[dst_v], add=True)
        pltpu.sync_copy(ones_v, cntacc_sh.at[dst_v], add=True)
        return carry
    lax.fori_loop(0, N_CHUNK, body, 0)
    plsc.subcore_barrier()

    out_base = c * N_NODES + base_r
    for k in range(ROWS_PER_TILE // ZCH):
        sl = pl.ds(base_r + k * ZCH, ZCH)
        osl = pl.ds(out_base + k * ZCH, ZCH)
        pltpu.sync_copy(efacc_sh.at[sl], zbuf)
        pltpu.sync_copy(zbuf, efout_hbm.at[osl])
        pltpu.sync_copy(cntacc_sh.at[sl], zbuf)
        pltpu.sync_copy(zbuf, cntout_hbm.at[osl])


def _matmul_relu_body(x_ref, w_ref, b_ref, o_ref):
    y = jnp.dot(x_ref[...], w_ref[...], preferred_element_type=jnp.float32)
    o_ref[...] = jnp.maximum(y + b_ref[...], 0.0)


def _ef_tc(ef_pack, w_bd, b_tile):
    rows = ef_pack.shape[0]
    blk = 4000
    grid = rows // blk
    return pl.pallas_call(
        _matmul_relu_body,
        grid=(grid,),
        in_specs=[
            pl.BlockSpec((blk, 128), lambda i: (i, 0)),
            pl.BlockSpec((128, 128), lambda i: (0, 0)),
            pl.BlockSpec((1, 128), lambda i: (0, 0)),
        ],
        out_specs=pl.BlockSpec((blk, 128), lambda i: (i, 0)),
        out_shape=jax.ShapeDtypeStruct((rows, 128), jnp.float32),
    )(ef_pack, w_bd, b_tile)


def _layer_body(a_ref, e_ref, c_ref, wh_ref, we_ref, b_ref, o_ref):
    agg = a_ref[0] + a_ref[1]
    efa = e_ref[0] + e_ref[1]
    cnt = (c_ref[0] + c_ref[1])[:, 0:1]
    inv = 1.0 / jnp.maximum(cnt, 1.0)
    y = (jnp.dot(agg, wh_ref[...], preferred_element_type=jnp.float32)
         + jnp.dot(efa, we_ref[...], preferred_element_type=jnp.float32)
         + cnt * b_ref[...])
    o_ref[...] = jnp.maximum(y * inv, 0.0)


def _layer_tc(aggp, efp, cntp, Wh, We, b2d):
    blk = 2000
    grid = N_NODES // blk
    return pl.pallas_call(
        _layer_body,
        grid=(grid,),
        in_specs=[
            pl.BlockSpec((2, blk, NODE_DIM), lambda i: (0, i, 0)),
            pl.BlockSpec((2, blk, EDGE_EMB), lambda i: (0, i, 0)),
            pl.BlockSpec((2, blk, EDGE_EMB), lambda i: (0, i, 0)),
            pl.BlockSpec((NODE_DIM, NODE_DIM), lambda i: (0, 0)),
            pl.BlockSpec((EDGE_EMB, NODE_DIM), lambda i: (0, 0)),
            pl.BlockSpec((1, NODE_DIM), lambda i: (0, 0)),
        ],
        out_specs=pl.BlockSpec((blk, NODE_DIM), lambda i: (i, 0)),
        out_shape=jax.ShapeDtypeStruct((N_NODES, NODE_DIM), jnp.float32),
    )(aggp, efp, cntp, Wh, We, b2d)


def _readout_body(h_ref, b_ref, a_ref, wf1_ref, bf1_ref, wf2_ref, bf2_ref,
                  s1_ref, x1_ref, x2_ref, s1acc, c1acc, s2acc, c2acc):
    i = pl.program_id(0)
    blk = h_ref.shape[0]

    @pl.when(i == 0)
    def _init():
        s1acc[...] = jnp.zeros_like(s1acc)
        c1acc[...] = jnp.zeros_like(c1acc)
        s2acc[...] = jnp.zeros_like(s2acc)
        c2acc[...] = jnp.zeros_like(c2acc)

    batch = b_ref[0, 0, :]
    anchor = a_ref[0, 0, :]
    ga = lax.broadcasted_iota(jnp.int32, (NUM_GRAPHS, blk), 0)
    m_g = ga == batch[None, :]
    oh1 = jnp.where(m_g & (anchor[None, :] == 0), 1.0, 0.0)
    oh2 = jnp.where(m_g & (anchor[None, :] == 1), 1.0, 0.0)
    h = h_ref[...]
    ones = jnp.ones((blk, NODE_DIM), jnp.float32)
    s1acc[...] += jnp.dot(oh1, h, preferred_element_type=jnp.float32)
    c1acc[...] += jnp.dot(oh1, ones, preferred_element_type=jnp.float32)
    s2acc[...] += jnp.dot(oh2, h, preferred_element_type=jnp.float32)
    c2acc[...] += jnp.dot(oh2, ones, preferred_element_type=jnp.float32)

    @pl.when(i == pl.num_programs(0) - 1)
    def _final():
        x1 = s1acc[...] / jnp.maximum(c1acc[...], 1.0)
        x2 = s2acc[...] / jnp.maximum(c2acc[...], 1.0)
        xsub = x1 - x2
        t = jnp.maximum(
            jnp.dot(xsub, wf1_ref[...], preferred_element_type=jnp.float32)
            + bf1_ref[...], 0.0)
        s1_ref[...] = (jnp.dot(t, wf2_ref[...],
                               preferred_element_type=jnp.float32)
                       + bf2_ref[...])
        x1_ref[...] = x1
        x2_ref[...] = x2


def _readout_tc(h, batch3, anchor3, W_f1, b_f1_2d, W_f2p, b_f2p):
    blk = 2000
    grid = N_NODES // blk
    G = NUM_GRAPHS
    return pl.pallas_call(
        _readout_body,
        grid=(grid,),
        in_specs=[
            pl.BlockSpec((blk, NODE_DIM), lambda i: (i, 0)),
            pl.BlockSpec((1, 1, blk), lambda i: (i, 0, 0)),
            pl.BlockSpec((1, 1, blk), lambda i: (i, 0, 0)),
            pl.BlockSpec((NODE_DIM, NODE_DIM), lambda i: (0, 0)),
            pl.BlockSpec((1, NODE_DIM), lambda i: (0, 0)),
            pl.BlockSpec((NODE_DIM, NODE_DIM), lambda i: (0, 0)),
            pl.BlockSpec((1, NODE_DIM), lambda i: (0, 0)),
        ],
        out_specs=[
            pl.BlockSpec((G, NODE_DIM), lambda i: (0, 0)),
            pl.BlockSpec((G, NODE_DIM), lambda i: (0, 0)),
            pl.BlockSpec((G, NODE_DIM), lambda i: (0, 0)),
        ],
        out_shape=[
            jax.ShapeDtypeStruct((G, NODE_DIM), jnp.float32),
            jax.ShapeDtypeStruct((G, NODE_DIM), jnp.float32),
            jax.ShapeDtypeStruct((G, NODE_DIM), jnp.float32),
        ],
        scratch_shapes=[
            pltpu.VMEM((G, NODE_DIM), jnp.float32),
            pltpu.VMEM((G, NODE_DIM), jnp.float32),
            pltpu.VMEM((G, NODE_DIM), jnp.float32),
            pltpu.VMEM((G, NODE_DIM), jnp.float32),
        ],
    )(h, batch3, anchor3, W_f1, b_f1_2d, W_f2p, b_f2p)


def kernel(x, edge_index, edge_features, batch, anchor, num_graphs,
           W_ef, b_ef, conv_Ws, conv_bs, W_f1, b_f1, W_f2, b_f2):
    src = edge_index[0].astype(jnp.int32)
    dst = edge_index[1].astype(jnp.int32)

    w_bd = jax.scipy.linalg.block_diag(*([W_ef] * 8))
    b_tile = jnp.tile(b_ef, 8).reshape(1, 128)
    ef_pack = edge_features.reshape(N_EDGES // 8, 128)
    ef = _ef_tc(ef_pack, w_bd, b_tile).reshape(N_EDGES, EDGE_EMB)

    efp_flat, cntp_flat = _edge_agg_sc(ef, dst)
    efp = efp_flat.reshape(NC, N_NODES, EDGE_EMB)
    cntp = cntp_flat.reshape(NC, N_NODES, EDGE_EMB)

    h = x
    for W, b in zip(conv_Ws, conv_bs):
        Wh = W[:-EDGE_EMB, :]
        We = W[-EDGE_EMB:, :]
        aggp = _seg_sum_sc(h, src, dst).reshape(NC, N_NODES, NODE_DIM)
        h = _layer_tc(aggp, efp, cntp, Wh, We, b.reshape(1, NODE_DIM))

    batch3 = batch.astype(jnp.int32).reshape(N_NODES // 2000, 1, 2000)
    anchor3 = anchor.astype(jnp.int32).reshape(N_NODES // 2000, 1, 2000)
    W_f2p = jnp.pad(W_f2, ((0, 0), (0, NODE_DIM - 1)))
    b_f2p = jnp.pad(b_f2, (0, NODE_DIM - 1)).reshape(1, NODE_DIM)
    scores_m, x1, x2 = _readout_tc(h, batch3, anchor3, W_f1,
                                   b_f1.reshape(1, NODE_DIM), W_f2p, b_f2p)
    return (scores_m[:, 0], h, x1, x2)


# speedup vs baseline: 3.8863x; 3.8863x over previous
import functools
import jax
import jax.numpy as jnp
from jax import lax
from jax.experimental import pallas as pl
from jax.experimental.pallas import tpu as pltpu
from jax.experimental.pallas import tpu_sc as plsc

N_NODES = 10000
N_EDGES = 320000
NODE_DIM = 128
EDGE_EMB = 16
NUM_GRAPHS = 256

NC = 2
NS = 16
NW = NC * NS
CH = 80
PER_TILE = N_EDGES // NW
N_CHUNK = PER_TILE // CH
N_PAD = 10240
ROWS_PER_TILE = N_PAD // NS
ZCH = 128

_mesh = plsc.VectorSubcoreMesh(
    core_axis_name="c", subcore_axis_name="s", num_cores=NC, num_subcores=NS)


@functools.partial(
    pl.kernel, mesh=_mesh,
    out_type=jax.ShapeDtypeStruct((NC * N_PAD, NODE_DIM), jnp.float32),
    scratch_types=[
        pltpu.VMEM((CH,), jnp.int32),
        pltpu.VMEM((CH,), jnp.int32),
        pltpu.VMEM((CH, NODE_DIM), jnp.float32),
        pltpu.VMEM((ZCH, NODE_DIM), jnp.float32),
        pltpu.VMEM_SHARED((N_PAD, NODE_DIM), jnp.float32),
        pltpu.SemaphoreType.DMA,
    ])
def _seg_sum_sc(h_hbm, src_hbm, dst_hbm, out_hbm, src_v, dst_v, rows_v,
                zbuf, acc_sh, sem):
    c = lax.axis_index("c")
    s = lax.axis_index("s")
    wid = c * NS + s
    zero16 = jnp.zeros((16,), jnp.float32)

    def zrow(i, carry):
        for j in range(8):
            zbuf[i, pl.ds(j * 16, 16)] = zero16
        return carry
    lax.fori_loop(0, ZCH, zrow, 0)

    base_r = s * ROWS_PER_TILE
    for k in range(ROWS_PER_TILE // ZCH):
        pltpu.sync_copy(zbuf, acc_sh.at[pl.ds(base_r + k * ZCH, ZCH)])
    plsc.subcore_barrier()

    ebase = wid * PER_TILE

    def body(i, carry):
        off = ebase + i * CH
        pltpu.sync_copy(src_hbm.at[pl.ds(off, CH)], src_v)
        pltpu.sync_copy(dst_hbm.at[pl.ds(off, CH)], dst_v)
        pltpu.async_copy(h_hbm.at[src_v], rows_v, sem).wait()
        pltpu.sync_copy(rows_v, acc_sh.at[dst_v], add=True)
        return carry
    lax.fori_loop(0, N_CHUNK, body, 0)
    plsc.subcore_barrier()

    out_base = c * N_PAD + base_r
    for k in range(ROWS_PER_TILE // ZCH):
        pltpu.sync_copy(acc_sh.at[pl.ds(base_r + k * ZCH, ZCH)], zbuf)
        pltpu.sync_copy(zbuf, out_hbm.at[pl.ds(out_base + k * ZCH, ZCH)])


@functools.partial(
    pl.kernel, mesh=_mesh,
    out_type=jax.ShapeDtypeStruct((NC * N_PAD, NODE_DIM), jnp.float32),
    scratch_types=[
        pltpu.VMEM((CH,), jnp.int32),
        pltpu.VMEM((CH, NODE_DIM), jnp.float32),
        pltpu.VMEM((ZCH, NODE_DIM), jnp.float32),
        pltpu.VMEM_SHARED((N_PAD, NODE_DIM), jnp.float32),
        pltpu.SemaphoreType.DMA,
    ])
def _edge_agg_sc(ef_hbm, dst_hbm, out_hbm, dst_v, rows_v, zbuf, acc_sh, sem):
    c = lax.axis_index("c")
    s = lax.axis_index("s")
    wid = c * NS + s
    zero16 = jnp.zeros((16,), jnp.float32)

    def zrow(i, carry):
        for j in range(8):
            zbuf[i, pl.ds(j * 16, 16)] = zero16
        return carry
    lax.fori_loop(0, ZCH, zrow, 0)

    base_r = s * ROWS_PER_TILE
    for k in range(ROWS_PER_TILE // ZCH):
        pltpu.sync_copy(zbuf, acc_sh.at[pl.ds(base_r + k * ZCH, ZCH)])
    plsc.subcore_barrier()

    ebase = wid * PER_TILE

    def body(i, carry):
        off = ebase + i * CH
        pltpu.sync_copy(dst_hbm.at[pl.ds(off, CH)], dst_v)
        pltpu.sync_copy(ef_hbm.at[pl.ds(off, CH)], rows_v)
        pltpu.sync_copy(rows_v, acc_sh.at[dst_v], add=True)
        return carry
    lax.fori_loop(0, N_CHUNK, body, 0)
    plsc.subcore_barrier()

    out_base = c * N_PAD + base_r
    for k in range(ROWS_PER_TILE // ZCH):
        pltpu.sync_copy(acc_sh.at[pl.ds(base_r + k * ZCH, ZCH)], zbuf)
        pltpu.sync_copy(zbuf, out_hbm.at[pl.ds(out_base + k * ZCH, ZCH)])


def _matmul_relu_body(x_ref, w_ref, b_ref, o_ref):
    y = jnp.dot(x_ref[...], w_ref[...], preferred_element_type=jnp.float32,
            precision=lax.Precision.HIGHEST)
    o_ref[...] = jnp.maximum(y + b_ref[...], 0.0)


def _ef_tc(ef_raw, w_pad, b_pad):
    blk = 2000
    grid = N_EDGES // blk
    return pl.pallas_call(
        _matmul_relu_body,
        grid=(grid,),
        in_specs=[
            pl.BlockSpec((blk, EDGE_EMB), lambda i: (i, 0)),
            pl.BlockSpec((EDGE_EMB, 128), lambda i: (0, 0)),
            pl.BlockSpec((1, 128), lambda i: (0, 0)),
        ],
        out_specs=pl.BlockSpec((blk, 128), lambda i: (i, 0)),
        out_shape=jax.ShapeDtypeStruct((N_EDGES, 128), jnp.float32),
    )(ef_raw, w_pad, b_pad)


def _layer_body(a_ref, e_ref, wh_ref, we_ref, b_ref, o_ref):
    agg = a_ref[0] + a_ref[1]
    ec = e_ref[0] + e_ref[1]
    efa = ec[:, 0:EDGE_EMB]
    cnt = ec[:, EDGE_EMB:EDGE_EMB + 1]
    inv = 1.0 / jnp.maximum(cnt, 1.0)
    y = (jnp.dot(agg, wh_ref[...], preferred_element_type=jnp.float32,
            precision=lax.Precision.HIGHEST)
         + jnp.dot(efa, we_ref[...], preferred_element_type=jnp.float32,
            precision=lax.Precision.HIGHEST)
         + cnt * b_ref[...])
    o_ref[...] = jnp.maximum(y * inv, 0.0)


def _layer_tc(aggp, ecp, Wh, We, b2d):
    blk = 2000
    grid = N_NODES // blk
    return pl.pallas_call(
        _layer_body,
        grid=(grid,),
        in_specs=[
            pl.BlockSpec((2, blk, NODE_DIM), lambda i: (0, i, 0)),
            pl.BlockSpec((2, blk, NODE_DIM), lambda i: (0, i, 0)),
            pl.BlockSpec((NODE_DIM, NODE_DIM), lambda i: (0, 0)),
            pl.BlockSpec((EDGE_EMB, NODE_DIM), lambda i: (0, 0)),
            pl.BlockSpec((1, NODE_DIM), lambda i: (0, 0)),
        ],
        out_specs=pl.BlockSpec((blk, NODE_DIM), lambda i: (i, 0)),
        out_shape=jax.ShapeDtypeStruct((N_NODES, NODE_DIM), jnp.float32),
    )(aggp, ecp, Wh, We, b2d)


def _readout_body(h_ref, b_ref, a_ref, wf1_ref, bf1_ref, wf2_ref, bf2_ref,
                  s1_ref, x1_ref, x2_ref, s1acc, c1acc, s2acc, c2acc):
    i = pl.program_id(0)
    blk = h_ref.shape[0]

    @pl.when(i == 0)
    def _init():
        s1acc[...] = jnp.zeros_like(s1acc)
        c1acc[...] = jnp.zeros_like(c1acc)
        s2acc[...] = jnp.zeros_like(s2acc)
        c2acc[...] = jnp.zeros_like(c2acc)

    batch = b_ref[0, 0, :]
    anchor = a_ref[0, 0, :]
    ga = lax.broadcasted_iota(jnp.int32, (NUM_GRAPHS, blk), 0)
    m_g = ga == batch[None, :]
    oh1 = jnp.where(m_g & (anchor[None, :] == 0), 1.0, 0.0)
    oh2 = jnp.where(m_g & (anchor[None, :] == 1), 1.0, 0.0)
    h = h_ref[...]
    ones = jnp.ones((blk, NODE_DIM), jnp.float32)
    s1acc[...] += jnp.dot(oh1, h, preferred_element_type=jnp.float32,
            precision=lax.Precision.HIGHEST)
    c1acc[...] += jnp.dot(oh1, ones, preferred_element_type=jnp.float32,
            precision=lax.Precision.HIGHEST)
    s2acc[...] += jnp.dot(oh2, h, preferred_element_type=jnp.float32,
            precision=lax.Precision.HIGHEST)
    c2acc[...] += jnp.dot(oh2, ones, preferred_element_type=jnp.float32,
            precision=lax.Precision.HIGHEST)

    @pl.when(i == pl.num_programs(0) - 1)
    def _final():
        x1 = s1acc[...] / jnp.maximum(c1acc[...], 1.0)
        x2 = s2acc[...] / jnp.maximum(c2acc[...], 1.0)
        xsub = x1 - x2
        t = jnp.maximum(
            jnp.dot(xsub, wf1_ref[...], preferred_element_type=jnp.float32,
            precision=lax.Precision.HIGHEST)
            + bf1_ref[...], 0.0)
        s1_ref[...] = (jnp.dot(t, wf2_ref[...],
                               preferred_element_type=jnp.float32,
            precision=lax.Precision.HIGHEST)
                       + bf2_ref[...])
        x1_ref[...] = x1
        x2_ref[...] = x2


def _readout_tc(h, batch3, anchor3, W_f1, b_f1_2d, W_f2p, b_f2p):
    blk = 2000
    grid = N_NODES // blk
    G = NUM_GRAPHS
    return pl.pallas_call(
        _readout_body,
        grid=(grid,),
        in_specs=[
            pl.BlockSpec((blk, NODE_DIM), lambda i: (i, 0)),
            pl.BlockSpec((1, 1, blk), lambda i: (i, 0, 0)),
            pl.BlockSpec((1, 1, blk), lambda i: (i, 0, 0)),
            pl.BlockSpec((NODE_DIM, NODE_DIM), lambda i: (0, 0)),
            pl.BlockSpec((1, NODE_DIM), lambda i: (0, 0)),
            pl.BlockSpec((NODE_DIM, NODE_DIM), lambda i: (0, 0)),
            pl.BlockSpec((1, NODE_DIM), lambda i: (0, 0)),
        ],
        out_specs=[
            pl.BlockSpec((G, NODE_DIM), lambda i: (0, 0)),
            pl.BlockSpec((G, NODE_DIM), lambda i: (0, 0)),
            pl.BlockSpec((G, NODE_DIM), lambda i: (0, 0)),
        ],
        out_shape=[
            jax.ShapeDtypeStruct((G, NODE_DIM), jnp.float32),
            jax.ShapeDtypeStruct((G, NODE_DIM), jnp.float32),
            jax.ShapeDtypeStruct((G, NODE_DIM), jnp.float32),
        ],
        scratch_shapes=[
            pltpu.VMEM((G, NODE_DIM), jnp.float32),
            pltpu.VMEM((G, NODE_DIM), jnp.float32),
            pltpu.VMEM((G, NODE_DIM), jnp.float32),
            pltpu.VMEM((G, NODE_DIM), jnp.float32),
        ],
    )(h, batch3, anchor3, W_f1, b_f1_2d, W_f2p, b_f2p)


def kernel(x, edge_index, edge_features, batch, anchor, num_graphs,
           W_ef, b_ef, conv_Ws, conv_bs, W_f1, b_f1, W_f2, b_f2):
    src = edge_index[0].astype(jnp.int32)
    dst = edge_index[1].astype(jnp.int32)

    w_pad = jnp.pad(W_ef, ((0, 0), (0, 128 - EDGE_EMB)))
    b_pad = jnp.concatenate(
        [b_ef, jnp.ones((EDGE_EMB,), jnp.float32),
         jnp.zeros((128 - 2 * EDGE_EMB,), jnp.float32)]).reshape(1, 128)
    ef128 = _ef_tc(edge_features, w_pad, b_pad)

    ecp = _edge_agg_sc(ef128, dst).reshape(NC, N_PAD, NODE_DIM)[:, :N_NODES]

    h = x
    for W, b in zip(conv_Ws, conv_bs):
        Wh = W[:-EDGE_EMB, :]
        We = W[-EDGE_EMB:, :]
        aggp = _seg_sum_sc(h, src, dst).reshape(NC, N_PAD, NODE_DIM)[:, :N_NODES]
        h = _layer_tc(aggp, ecp, Wh, We, b.reshape(1, NODE_DIM))

    batch3 = batch.astype(jnp.int32).reshape(N_NODES // 2000, 1, 2000)
    anchor3 = anchor.astype(jnp.int32).reshape(N_NODES // 2000, 1, 2000)
    W_f2p = jnp.pad(W_f2, ((0, 0), (0, NODE_DIM - 1)))
    b_f2p = jnp.pad(b_f2, (0, NODE_DIM - 1)).reshape(1, NODE_DIM)
    scores_m, x1, x2 = _readout_tc(h, batch3, anchor3, W_f1,
                                   b_f1.reshape(1, NODE_DIM), W_f2p, b_f2p)
    return (scores_m[:, 0], h, x1, x2)


# speedup vs baseline: 5.7789x; 1.4870x over previous
import functools
import jax
import jax.numpy as jnp
from jax import lax
from jax.experimental import pallas as pl
from jax.experimental.pallas import tpu as pltpu
from jax.experimental.pallas import tpu_sc as plsc

N_NODES = 10000
N_EDGES = 320000
NODE_DIM = 128
EDGE_EMB = 16
NUM_GRAPHS = 256

NC = 2
NS = 16
NW = NC * NS
CH = 80
PER_TILE = N_EDGES // NW
N_CHUNK = PER_TILE // CH
N_PAD = 10240
ROWS_PER_TILE = N_PAD // NS
ZCH = 64

_mesh = plsc.VectorSubcoreMesh(
    core_axis_name="c", subcore_axis_name="s", num_cores=NC, num_subcores=NS)


NB = 3


@functools.partial(
    pl.kernel, mesh=_mesh,
    out_type=jax.ShapeDtypeStruct((NC * N_PAD, NODE_DIM), jnp.float32),
    scratch_types=[
        pltpu.VMEM((CH,), jnp.int32),
        pltpu.VMEM((CH,), jnp.int32),
        pltpu.VMEM((CH,), jnp.int32),
        pltpu.VMEM((CH,), jnp.int32),
        pltpu.VMEM((CH, NODE_DIM), jnp.float32),
        pltpu.VMEM((CH, NODE_DIM), jnp.float32),
        pltpu.VMEM((CH, NODE_DIM), jnp.float32),
        pltpu.VMEM((ZCH, NODE_DIM), jnp.float32),
        pltpu.VMEM_SHARED((N_PAD, NODE_DIM), jnp.float32),
        pltpu.SemaphoreType.DMA,
        pltpu.SemaphoreType.DMA,
        pltpu.SemaphoreType.DMA,
    ])
def _seg_sum_sc(h_hbm, src_hbm, dst_hbm, out_hbm, s0, s1, s2, dst_v,
                r0, r1, r2, zbuf, acc_sh, g0, g1, g2):
    c = lax.axis_index("c")
    s = lax.axis_index("s")
    wid = c * NS + s
    srcb = [s0, s1, s2]
    rowb = [r0, r1, r2]
    semb = [g0, g1, g2]
    zero16 = jnp.zeros((16,), jnp.float32)

    def zrow(i, carry):
        for j in range(8):
            zbuf[i, pl.ds(j * 16, 16)] = zero16
        return carry
    lax.fori_loop(0, ZCH, zrow, 0)

    base_r = s * ROWS_PER_TILE
    for k in range(ROWS_PER_TILE // ZCH):
        pltpu.sync_copy(zbuf, acc_sh.at[pl.ds(base_r + k * ZCH, ZCH)])
    plsc.subcore_barrier()

    ebase = wid * PER_TILE

    pltpu.sync_copy(src_hbm.at[pl.ds(ebase, CH)], s0)
    pltpu.async_copy(h_hbm.at[s0], r0, g0)
    pltpu.sync_copy(src_hbm.at[pl.ds(ebase + CH, CH)], s1)
    pltpu.async_copy(h_hbm.at[s1], r1, g1)

    def group(g, carry):
        for b in range(NB):
            v = g * NB + b
            bp = (b + 2) % NB

            @pl.when(v + 2 < N_CHUNK)
            def _prefetch():
                pltpu.sync_copy(
                    src_hbm.at[pl.ds(ebase + (v + 2) * CH, CH)], srcb[bp])
                pltpu.async_copy(h_hbm.at[srcb[bp]], rowb[bp], semb[bp])

            @pl.when(v < N_CHUNK)
            def _visit():
                pltpu.sync_copy(dst_hbm.at[pl.ds(ebase + v * CH, CH)], dst_v)
                pltpu.make_async_copy(h_hbm.at[srcb[b]], rowb[b],
                                      semb[b]).wait()
                pltpu.sync_copy(rowb[b], acc_sh.at[dst_v], add=True)
        return carry
    lax.fori_loop(0, (N_CHUNK + NB - 1) // NB, group, 0)
    plsc.subcore_barrier()

    out_base = c * N_PAD + base_r
    for k in range(ROWS_PER_TILE // ZCH):
        pltpu.sync_copy(acc_sh.at[pl.ds(base_r + k * ZCH, ZCH)], zbuf)
        pltpu.sync_copy(zbuf, out_hbm.at[pl.ds(out_base + k * ZCH, ZCH)])


@functools.partial(
    pl.kernel, mesh=_mesh,
    out_type=jax.ShapeDtypeStruct((NC * N_PAD, NODE_DIM), jnp.float32),
    scratch_types=[
        pltpu.VMEM((CH,), jnp.int32),
        pltpu.VMEM((CH, NODE_DIM), jnp.float32),
        pltpu.VMEM((CH, NODE_DIM), jnp.float32),
        pltpu.VMEM((CH, NODE_DIM), jnp.float32),
        pltpu.VMEM((ZCH, NODE_DIM), jnp.float32),
        pltpu.VMEM_SHARED((N_PAD, NODE_DIM), jnp.float32),
        pltpu.SemaphoreType.DMA,
        pltpu.SemaphoreType.DMA,
        pltpu.SemaphoreType.DMA,
    ])
def _edge_agg_sc(ef_hbm, dst_hbm, out_hbm, dst_v, r0, r1, r2, zbuf, acc_sh,
                 g0, g1, g2):
    c = lax.axis_index("c")
    s = lax.axis_index("s")
    wid = c * NS + s
    rowb = [r0, r1, r2]
    semb = [g0, g1, g2]
    zero16 = jnp.zeros((16,), jnp.float32)

    def zrow(i, carry):
        for j in range(8):
            zbuf[i, pl.ds(j * 16, 16)] = zero16
        return carry
    lax.fori_loop(0, ZCH, zrow, 0)

    base_r = s * ROWS_PER_TILE
    for k in range(ROWS_PER_TILE // ZCH):
        pltpu.sync_copy(zbuf, acc_sh.at[pl.ds(base_r + k * ZCH, ZCH)])
    plsc.subcore_barrier()

    ebase = wid * PER_TILE

    pltpu.async_copy(ef_hbm.at[pl.ds(ebase, CH)], r0, g0)
    pltpu.async_copy(ef_hbm.at[pl.ds(ebase + CH, CH)], r1, g1)

    def group(g, carry):
        for b in range(NB):
            v = g * NB + b
            bp = (b + 2) % NB

            @pl.when(v + 2 < N_CHUNK)
            def _prefetch():
                pltpu.async_copy(
                    ef_hbm.at[pl.ds(ebase + (v + 2) * CH, CH)], rowb[bp],
                    semb[bp])

            @pl.when(v < N_CHUNK)
            def _visit():
                pltpu.sync_copy(dst_hbm.at[pl.ds(ebase + v * CH, CH)], dst_v)
                pltpu.make_async_copy(
                    ef_hbm.at[pl.ds(ebase, CH)], rowb[b], semb[b]).wait()
                pltpu.sync_copy(rowb[b], acc_sh.at[dst_v], add=True)
        return carry
    lax.fori_loop(0, (N_CHUNK + NB - 1) // NB, group, 0)
    plsc.subcore_barrier()

    out_base = c * N_PAD + base_r
    for k in range(ROWS_PER_TILE // ZCH):
        pltpu.sync_copy(acc_sh.at[pl.ds(base_r + k * ZCH, ZCH)], zbuf)
        pltpu.sync_copy(zbuf, out_hbm.at[pl.ds(out_base + k * ZCH, ZCH)])


def _matmul_relu_body(x_ref, w_ref, b_ref, o_ref):
    y = jnp.dot(x_ref[...], w_ref[...], preferred_element_type=jnp.float32,
            precision=lax.Precision.HIGHEST)
    o_ref[...] = jnp.maximum(y + b_ref[...], 0.0)


def _ef_tc(ef_raw, w_pad, b_pad):
    blk = 2000
    grid = N_EDGES // blk
    return pl.pallas_call(
        _matmul_relu_body,
        grid=(grid,),
        in_specs=[
            pl.BlockSpec((blk, EDGE_EMB), lambda i: (i, 0)),
            pl.BlockSpec((EDGE_EMB, 128), lambda i: (0, 0)),
            pl.BlockSpec((1, 128), lambda i: (0, 0)),
        ],
        out_specs=pl.BlockSpec((blk, 128), lambda i: (i, 0)),
        out_shape=jax.ShapeDtypeStruct((N_EDGES, 128), jnp.float32),
    )(ef_raw, w_pad, b_pad)


def _layer_body(a_ref, e_ref, wh_ref, we_ref, b_ref, o_ref):
    agg = a_ref[0] + a_ref[1]
    ec = e_ref[0] + e_ref[1]
    efa = ec[:, 0:EDGE_EMB]
    cnt = ec[:, EDGE_EMB:EDGE_EMB + 1]
    inv = 1.0 / jnp.maximum(cnt, 1.0)
    y = (jnp.dot(agg, wh_ref[...], preferred_element_type=jnp.float32,
            precision=lax.Precision.HIGHEST)
         + jnp.dot(efa, we_ref[...], preferred_element_type=jnp.float32,
            precision=lax.Precision.HIGHEST)
         + cnt * b_ref[...])
    o_ref[...] = jnp.maximum(y * inv, 0.0)


def _layer_tc(aggp, ecp, Wh, We, b2d):
    blk = 2000
    grid = N_NODES // blk
    return pl.pallas_call(
        _layer_body,
        grid=(grid,),
        in_specs=[
            pl.BlockSpec((2, blk, NODE_DIM), lambda i: (0, i, 0)),
            pl.BlockSpec((2, blk, NODE_DIM), lambda i: (0, i, 0)),
            pl.BlockSpec((NODE_DIM, NODE_DIM), lambda i: (0, 0)),
            pl.BlockSpec((EDGE_EMB, NODE_DIM), lambda i: (0, 0)),
            pl.BlockSpec((1, NODE_DIM), lambda i: (0, 0)),
        ],
        out_specs=pl.BlockSpec((blk, NODE_DIM), lambda i: (i, 0)),
        out_shape=jax.ShapeDtypeStruct((N_NODES, NODE_DIM), jnp.float32),
    )(aggp, ecp, Wh, We, b2d)


def _readout_body(h_ref, b_ref, a_ref, wf1_ref, bf1_ref, wf2_ref, bf2_ref,
                  s1_ref, x1_ref, x2_ref, s1acc, c1acc, s2acc, c2acc):
    i = pl.program_id(0)
    blk = h_ref.shape[0]

    @pl.when(i == 0)
    def _init():
        s1acc[...] = jnp.zeros_like(s1acc)
        c1acc[...] = jnp.zeros_like(c1acc)
        s2acc[...] = jnp.zeros_like(s2acc)
        c2acc[...] = jnp.zeros_like(c2acc)

    batch = b_ref[0, 0, :]
    anchor = a_ref[0, 0, :]
    ga = lax.broadcasted_iota(jnp.int32, (NUM_GRAPHS, blk), 0)
    m_g = ga == batch[None, :]
    oh1 = jnp.where(m_g & (anchor[None, :] == 0), 1.0, 0.0)
    oh2 = jnp.where(m_g & (anchor[None, :] == 1), 1.0, 0.0)
    h = h_ref[...]
    ones = jnp.ones((blk, NODE_DIM), jnp.float32)
    s1acc[...] += jnp.dot(oh1, h, preferred_element_type=jnp.float32,
            precision=lax.Precision.HIGHEST)
    c1acc[...] += jnp.dot(oh1, ones, preferred_element_type=jnp.float32,
            precision=lax.Precision.HIGHEST)
    s2acc[...] += jnp.dot(oh2, h, preferred_element_type=jnp.float32,
            precision=lax.Precision.HIGHEST)
    c2acc[...] += jnp.dot(oh2, ones, preferred_element_type=jnp.float32,
            precision=lax.Precision.HIGHEST)

    @pl.when(i == pl.num_programs(0) - 1)
    def _final():
        x1 = s1acc[...] / jnp.maximum(c1acc[...], 1.0)
        x2 = s2acc[...] / jnp.maximum(c2acc[...], 1.0)
        xsub = x1 - x2
        t = jnp.maximum(
            jnp.dot(xsub, wf1_ref[...], preferred_element_type=jnp.float32,
            precision=lax.Precision.HIGHEST)
            + bf1_ref[...], 0.0)
        s1_ref[...] = (jnp.dot(t, wf2_ref[...],
                               preferred_element_type=jnp.float32,
            precision=lax.Precision.HIGHEST)
                       + bf2_ref[...])
        x1_ref[...] = x1
        x2_ref[...] = x2


def _readout_tc(h, batch3, anchor3, W_f1, b_f1_2d, W_f2p, b_f2p):
    blk = 2000
    grid = N_NODES // blk
    G = NUM_GRAPHS
    return pl.pallas_call(
        _readout_body,
        grid=(grid,),
        in_specs=[
            pl.BlockSpec((blk, NODE_DIM), lambda i: (i, 0)),
            pl.BlockSpec((1, 1, blk), lambda i: (i, 0, 0)),
            pl.BlockSpec((1, 1, blk), lambda i: (i, 0, 0)),
            pl.BlockSpec((NODE_DIM, NODE_DIM), lambda i: (0, 0)),
            pl.BlockSpec((1, NODE_DIM), lambda i: (0, 0)),
            pl.BlockSpec((NODE_DIM, NODE_DIM), lambda i: (0, 0)),
            pl.BlockSpec((1, NODE_DIM), lambda i: (0, 0)),
        ],
        out_specs=[
            pl.BlockSpec((G, NODE_DIM), lambda i: (0, 0)),
            pl.BlockSpec((G, NODE_DIM), lambda i: (0, 0)),
            pl.BlockSpec((G, NODE_DIM), lambda i: (0, 0)),
        ],
        out_shape=[
            jax.ShapeDtypeStruct((G, NODE_DIM), jnp.float32),
            jax.ShapeDtypeStruct((G, NODE_DIM), jnp.float32),
            jax.ShapeDtypeStruct((G, NODE_DIM), jnp.float32),
        ],
        scratch_shapes=[
            pltpu.VMEM((G, NODE_DIM), jnp.float32),
            pltpu.VMEM((G, NODE_DIM), jnp.float32),
            pltpu.VMEM((G, NODE_DIM), jnp.float32),
            pltpu.VMEM((G, NODE_DIM), jnp.float32),
        ],
    )(h, batch3, anchor3, W_f1, b_f1_2d, W_f2p, b_f2p)


def kernel(x, edge_index, edge_features, batch, anchor, num_graphs,
           W_ef, b_ef, conv_Ws, conv_bs, W_f1, b_f1, W_f2, b_f2):
    src = edge_index[0].astype(jnp.int32)
    dst = edge_index[1].astype(jnp.int32)

    w_pad = jnp.pad(W_ef, ((0, 0), (0, 128 - EDGE_EMB)))
    b_pad = jnp.concatenate(
        [b_ef, jnp.ones((EDGE_EMB,), jnp.float32),
         jnp.zeros((128 - 2 * EDGE_EMB,), jnp.float32)]).reshape(1, 128)
    ef128 = _ef_tc(edge_features, w_pad, b_pad)

    ecp = _edge_agg_sc(ef128, dst).reshape(NC, N_PAD, NODE_DIM)[:, :N_NODES]

    h = x
    for W, b in zip(conv_Ws, conv_bs):
        Wh = W[:-EDGE_EMB, :]
        We = W[-EDGE_EMB:, :]
        aggp = _seg_sum_sc(h, src, dst).reshape(NC, N_PAD, NODE_DIM)[:, :N_NODES]
        h = _layer_tc(aggp, ecp, Wh, We, b.reshape(1, NODE_DIM))

    batch3 = batch.astype(jnp.int32).reshape(N_NODES // 2000, 1, 2000)
    anchor3 = anchor.astype(jnp.int32).reshape(N_NODES // 2000, 1, 2000)
    W_f2p = jnp.pad(W_f2, ((0, 0), (0, NODE_DIM - 1)))
    b_f2p = jnp.pad(b_f2, (0, NODE_DIM - 1)).reshape(1, NODE_DIM)
    scores_m, x1, x2 = _readout_tc(h, batch3, anchor3, W_f1,
                                   b_f1.reshape(1, NODE_DIM), W_f2p, b_f2p)
    return (scores_m[:, 0], h, x1, x2)


# speedup vs baseline: 6.8420x; 1.1840x over previous
import functools
import jax
import jax.numpy as jnp
from jax import lax
from jax.experimental import pallas as pl
from jax.experimental.pallas import tpu as pltpu
from jax.experimental.pallas import tpu_sc as plsc

N_NODES = 10000
N_EDGES = 320000
NODE_DIM = 128
EDGE_EMB = 16
NUM_GRAPHS = 256

NC = 2
NS = 16
NW = NC * NS
CH = 80
PER_TILE = N_EDGES // NW
N_CHUNK = PER_TILE // CH
N_PAD = 10240
ROWS_PER_TILE = N_PAD // NS
ZCH = 64

_mesh = plsc.VectorSubcoreMesh(
    core_axis_name="c", subcore_axis_name="s", num_cores=NC, num_subcores=NS)


NB = 3


@functools.partial(
    pl.kernel, mesh=_mesh,
    out_type=jax.ShapeDtypeStruct((NC * N_PAD, NODE_DIM), jnp.float32),
    scratch_types=[
        pltpu.VMEM((CH,), jnp.int32),
        pltpu.VMEM((CH,), jnp.int32),
        pltpu.VMEM((CH,), jnp.int32),
        pltpu.VMEM((CH,), jnp.int32),
        pltpu.VMEM((CH,), jnp.int32),
        pltpu.VMEM((CH,), jnp.int32),
        pltpu.VMEM((CH, NODE_DIM), jnp.float32),
        pltpu.VMEM((CH, NODE_DIM), jnp.float32),
        pltpu.VMEM((CH, NODE_DIM), jnp.float32),
        pltpu.VMEM((ZCH, NODE_DIM), jnp.float32),
        pltpu.VMEM_SHARED((N_PAD, NODE_DIM), jnp.float32),
        pltpu.SemaphoreType.DMA,
        pltpu.SemaphoreType.DMA,
        pltpu.SemaphoreType.DMA,
        pltpu.SemaphoreType.DMA,
        pltpu.SemaphoreType.DMA,
        pltpu.SemaphoreType.DMA,
    ])
def _seg_sum_sc(h_hbm, src_hbm, dst_hbm, out_hbm, s0, s1, s2, d0, d1, d2,
                r0, r1, r2, zbuf, acc_sh, g0, g1, g2, q0, q1, q2):
    c = lax.axis_index("c")
    s = lax.axis_index("s")
    wid = c * NS + s
    srcb = [s0, s1, s2]
    dstb = [d0, d1, d2]
    rowb = [r0, r1, r2]
    semb = [g0, g1, g2]
    dsem = [q0, q1, q2]
    zero16 = jnp.zeros((16,), jnp.float32)

    def zrow(i, carry):
        for j in range(8):
            zbuf[i, pl.ds(j * 16, 16)] = zero16
        return carry
    lax.fori_loop(0, ZCH, zrow, 0)

    base_r = s * ROWS_PER_TILE
    for k in range(ROWS_PER_TILE // ZCH):
        pltpu.sync_copy(zbuf, acc_sh.at[pl.ds(base_r + k * ZCH, ZCH)])
    plsc.subcore_barrier()

    ebase = wid * PER_TILE

    pltpu.sync_copy(src_hbm.at[pl.ds(ebase, CH)], s0)
    pltpu.async_copy(h_hbm.at[s0], r0, g0)
    pltpu.async_copy(dst_hbm.at[pl.ds(ebase, CH)], d0, q0)
    pltpu.sync_copy(src_hbm.at[pl.ds(ebase + CH, CH)], s1)
    pltpu.async_copy(h_hbm.at[s1], r1, g1)
    pltpu.async_copy(dst_hbm.at[pl.ds(ebase + CH, CH)], d1, q1)

    def group(g, carry):
        for b in range(NB):
            v = g * NB + b
            bp = (b + 2) % NB

            @pl.when(v + 2 < N_CHUNK)
            def _prefetch():
                pltpu.sync_copy(
                    src_hbm.at[pl.ds(ebase + (v + 2) * CH, CH)], srcb[bp])
                pltpu.async_copy(h_hbm.at[srcb[bp]], rowb[bp], semb[bp])
                pltpu.async_copy(
                    dst_hbm.at[pl.ds(ebase + (v + 2) * CH, CH)], dstb[bp],
                    dsem[bp])

            @pl.when(v < N_CHUNK)
            def _visit():
                pltpu.make_async_copy(
                    dst_hbm.at[pl.ds(ebase, CH)], dstb[b], dsem[b]).wait()
                pltpu.make_async_copy(h_hbm.at[srcb[b]], rowb[b],
                                      semb[b]).wait()
                pltpu.sync_copy(rowb[b], acc_sh.at[dstb[b]], add=True)
        return carry
    lax.fori_loop(0, (N_CHUNK + NB - 1) // NB, group, 0)
    plsc.subcore_barrier()

    out_base = c * N_PAD + base_r
    for k in range(ROWS_PER_TILE // ZCH):
        pltpu.sync_copy(acc_sh.at[pl.ds(base_r + k * ZCH, ZCH)], zbuf)
        pltpu.sync_copy(zbuf, out_hbm.at[pl.ds(out_base + k * ZCH, ZCH)])


@functools.partial(
    pl.kernel, mesh=_mesh,
    out_type=jax.ShapeDtypeStruct((NC * N_PAD, NODE_DIM), jnp.float32),
    scratch_types=[
        pltpu.VMEM((CH,), jnp.int32),
        pltpu.VMEM((CH,), jnp.int32),
        pltpu.VMEM((CH,), jnp.int32),
        pltpu.VMEM((CH, NODE_DIM), jnp.float32),
        pltpu.VMEM((CH, NODE_DIM), jnp.float32),
        pltpu.VMEM((CH, NODE_DIM), jnp.float32),
        pltpu.VMEM((ZCH, NODE_DIM), jnp.float32),
        pltpu.VMEM_SHARED((N_PAD, NODE_DIM), jnp.float32),
        pltpu.SemaphoreType.DMA,
        pltpu.SemaphoreType.DMA,
        pltpu.SemaphoreType.DMA,
        pltpu.SemaphoreType.DMA,
        pltpu.SemaphoreType.DMA,
        pltpu.SemaphoreType.DMA,
    ])
def _edge_agg_sc(ef_hbm, dst_hbm, out_hbm, d0, d1, d2, r0, r1, r2, zbuf,
                 acc_sh, g0, g1, g2, q0, q1, q2):
    c = lax.axis_index("c")
    s = lax.axis_index("s")
    wid = c * NS + s
    dstb = [d0, d1, d2]
    rowb = [r0, r1, r2]
    semb = [g0, g1, g2]
    dsem = [q0, q1, q2]
    zero16 = jnp.zeros((16,), jnp.float32)

    def zrow(i, carry):
        for j in range(8):
            zbuf[i, pl.ds(j * 16, 16)] = zero16
        return carry
    lax.fori_loop(0, ZCH, zrow, 0)

    base_r = s * ROWS_PER_TILE
    for k in range(ROWS_PER_TILE // ZCH):
        pltpu.sync_copy(zbuf, acc_sh.at[pl.ds(base_r + k * ZCH, ZCH)])
    plsc.subcore_barrier()

    ebase = wid * PER_TILE

    pltpu.async_copy(ef_hbm.at[pl.ds(ebase, CH)], r0, g0)
    pltpu.async_copy(dst_hbm.at[pl.ds(ebase, CH)], d0, q0)
    pltpu.async_copy(ef_hbm.at[pl.ds(ebase + CH, CH)], r1, g1)
    pltpu.async_copy(dst_hbm.at[pl.ds(ebase + CH, CH)], d1, q1)

    def group(g, carry):
        for b in range(NB):
            v = g * NB + b
            bp = (b + 2) % NB

            @pl.when(v + 2 < N_CHUNK)
            def _prefetch():
                pltpu.async_copy(
                    ef_hbm.at[pl.ds(ebase + (v + 2) * CH, CH)], rowb[bp],
                    semb[bp])
                pltpu.async_copy(
                    dst_hbm.at[pl.ds(ebase + (v + 2) * CH, CH)], dstb[bp],
                    dsem[bp])

            @pl.when(v < N_CHUNK)
            def _visit():
                pltpu.make_async_copy(
                    dst_hbm.at[pl.ds(ebase, CH)], dstb[b], dsem[b]).wait()
                pltpu.make_async_copy(
                    ef_hbm.at[pl.ds(ebase, CH)], rowb[b], semb[b]).wait()
                pltpu.sync_copy(rowb[b], acc_sh.at[dstb[b]], add=True)
        return carry
    lax.fori_loop(0, (N_CHUNK + NB - 1) // NB, group, 0)
    plsc.subcore_barrier()

    out_base = c * N_PAD + base_r
    for k in range(ROWS_PER_TILE // ZCH):
        pltpu.sync_copy(acc_sh.at[pl.ds(base_r + k * ZCH, ZCH)], zbuf)
        pltpu.sync_copy(zbuf, out_hbm.at[pl.ds(out_base + k * ZCH, ZCH)])


def _matmul_relu_body(x_ref, w_ref, b_ref, o_ref):
    y = jnp.dot(x_ref[...], w_ref[...], preferred_element_type=jnp.float32,
            precision=lax.Precision.HIGHEST)
    o_ref[...] = jnp.maximum(y + b_ref[...], 0.0)


def _ef_tc(ef_raw, w_pad, b_pad):
    blk = 2000
    grid = N_EDGES // blk
    return pl.pallas_call(
        _matmul_relu_body,
        grid=(grid,),
        in_specs=[
            pl.BlockSpec((blk, EDGE_EMB), lambda i: (i, 0)),
            pl.BlockSpec((EDGE_EMB, 128), lambda i: (0, 0)),
            pl.BlockSpec((1, 128), lambda i: (0, 0)),
        ],
        out_specs=pl.BlockSpec((blk, 128), lambda i: (i, 0)),
        out_shape=jax.ShapeDtypeStruct((N_EDGES, 128), jnp.float32),
    )(ef_raw, w_pad, b_pad)


def _layer_body(a_ref, e_ref, wh_ref, we_ref, b_ref, o_ref):
    agg = a_ref[0] + a_ref[1]
    ec = e_ref[0] + e_ref[1]
    efa = ec[:, 0:EDGE_EMB]
    cnt = ec[:, EDGE_EMB:EDGE_EMB + 1]
    inv = 1.0 / jnp.maximum(cnt, 1.0)
    y = (jnp.dot(agg, wh_ref[...], preferred_element_type=jnp.float32,
            precision=lax.Precision.HIGHEST)
         + jnp.dot(efa, we_ref[...], preferred_element_type=jnp.float32,
            precision=lax.Precision.HIGHEST)
         + cnt * b_ref[...])
    o_ref[...] = jnp.maximum(y * inv, 0.0)


def _layer_tc(aggp, ecp, Wh, We, b2d):
    blk = 2000
    grid = N_NODES // blk
    return pl.pallas_call(
        _layer_body,
        grid=(grid,),
        in_specs=[
            pl.BlockSpec((2, blk, NODE_DIM), lambda i: (0, i, 0)),
            pl.BlockSpec((2, blk, NODE_DIM), lambda i: (0, i, 0)),
            pl.BlockSpec((NODE_DIM, NODE_DIM), lambda i: (0, 0)),
            pl.BlockSpec((EDGE_EMB, NODE_DIM), lambda i: (0, 0)),
            pl.BlockSpec((1, NODE_DIM), lambda i: (0, 0)),
        ],
        out_specs=pl.BlockSpec((blk, NODE_DIM), lambda i: (i, 0)),
        out_shape=jax.ShapeDtypeStruct((N_NODES, NODE_DIM), jnp.float32),
    )(aggp, ecp, Wh, We, b2d)


def _readout_body(h_ref, b_ref, a_ref, wf1_ref, bf1_ref, wf2_ref, bf2_ref,
                  s1_ref, x1_ref, x2_ref, s1acc, c1acc, s2acc, c2acc):
    i = pl.program_id(0)
    blk = h_ref.shape[0]

    @pl.when(i == 0)
    def _init():
        s1acc[...] = jnp.zeros_like(s1acc)
        c1acc[...] = jnp.zeros_like(c1acc)
        s2acc[...] = jnp.zeros_like(s2acc)
        c2acc[...] = jnp.zeros_like(c2acc)

    batch = b_ref[0, 0, :]
    anchor = a_ref[0, 0, :]
    ga = lax.broadcasted_iota(jnp.int32, (NUM_GRAPHS, blk), 0)
    m_g = ga == batch[None, :]
    oh1 = jnp.where(m_g & (anchor[None, :] == 0), 1.0, 0.0)
    oh2 = jnp.where(m_g & (anchor[None, :] == 1), 1.0, 0.0)
    h = h_ref[...]
    ones = jnp.ones((blk, NODE_DIM), jnp.float32)
    s1acc[...] += jnp.dot(oh1, h, preferred_element_type=jnp.float32,
            precision=lax.Precision.HIGHEST)
    c1acc[...] += jnp.dot(oh1, ones, preferred_element_type=jnp.float32,
            precision=lax.Precision.HIGHEST)
    s2acc[...] += jnp.dot(oh2, h, preferred_element_type=jnp.float32,
            precision=lax.Precision.HIGHEST)
    c2acc[...] += jnp.dot(oh2, ones, preferred_element_type=jnp.float32,
            precision=lax.Precision.HIGHEST)

    @pl.when(i == pl.num_programs(0) - 1)
    def _final():
        x1 = s1acc[...] / jnp.maximum(c1acc[...], 1.0)
        x2 = s2acc[...] / jnp.maximum(c2acc[...], 1.0)
        xsub = x1 - x2
        t = jnp.maximum(
            jnp.dot(xsub, wf1_ref[...], preferred_element_type=jnp.float32,
            precision=lax.Precision.HIGHEST)
            + bf1_ref[...], 0.0)
        s1_ref[...] = (jnp.dot(t, wf2_ref[...],
                               preferred_element_type=jnp.float32,
            precision=lax.Precision.HIGHEST)
                       + bf2_ref[...])
        x1_ref[...] = x1
        x2_ref[...] = x2


def _readout_tc(h, batch3, anchor3, W_f1, b_f1_2d, W_f2p, b_f2p):
    blk = 2000
    grid = N_NODES // blk
    G = NUM_GRAPHS
    return pl.pallas_call(
        _readout_body,
        grid=(grid,),
        in_specs=[
            pl.BlockSpec((blk, NODE_DIM), lambda i: (i, 0)),
            pl.BlockSpec((1, 1, blk), lambda i: (i, 0, 0)),
            pl.BlockSpec((1, 1, blk), lambda i: (i, 0, 0)),
            pl.BlockSpec((NODE_DIM, NODE_DIM), lambda i: (0, 0)),
            pl.BlockSpec((1, NODE_DIM), lambda i: (0, 0)),
            pl.BlockSpec((NODE_DIM, NODE_DIM), lambda i: (0, 0)),
            pl.BlockSpec((1, NODE_DIM), lambda i: (0, 0)),
        ],
        out_specs=[
            pl.BlockSpec((G, NODE_DIM), lambda i: (0, 0)),
            pl.BlockSpec((G, NODE_DIM), lambda i: (0, 0)),
            pl.BlockSpec((G, NODE_DIM), lambda i: (0, 0)),
        ],
        out_shape=[
            jax.ShapeDtypeStruct((G, NODE_DIM), jnp.float32),
            jax.ShapeDtypeStruct((G, NODE_DIM), jnp.float32),
            jax.ShapeDtypeStruct((G, NODE_DIM), jnp.float32),
        ],
        scratch_shapes=[
            pltpu.VMEM((G, NODE_DIM), jnp.float32),
            pltpu.VMEM((G, NODE_DIM), jnp.float32),
            pltpu.VMEM((G, NODE_DIM), jnp.float32),
            pltpu.VMEM((G, NODE_DIM), jnp.float32),
        ],
    )(h, batch3, anchor3, W_f1, b_f1_2d, W_f2p, b_f2p)


def kernel(x, edge_index, edge_features, batch, anchor, num_graphs,
           W_ef, b_ef, conv_Ws, conv_bs, W_f1, b_f1, W_f2, b_f2):
    src = edge_index[0].astype(jnp.int32)
    dst = edge_index[1].astype(jnp.int32)

    w_pad = jnp.pad(W_ef, ((0, 0), (0, 128 - EDGE_EMB)))
    b_pad = jnp.concatenate(
        [b_ef, jnp.ones((EDGE_EMB,), jnp.float32),
         jnp.zeros((128 - 2 * EDGE_EMB,), jnp.float32)]).reshape(1, 128)
    ef128 = _ef_tc(edge_features, w_pad, b_pad)

    ecp = _edge_agg_sc(ef128, dst).reshape(NC, N_PAD, NODE_DIM)[:, :N_NODES]

    h = x
    for W, b in zip(conv_Ws, conv_bs):
        Wh = W[:-EDGE_EMB, :]
        We = W[-EDGE_EMB:, :]
        aggp = _seg_sum_sc(h, src, dst).reshape(NC, N_PAD, NODE_DIM)[:, :N_NODES]
        h = _layer_tc(aggp, ecp, Wh, We, b.reshape(1, NODE_DIM))

    batch3 = batch.astype(jnp.int32).reshape(N_NODES // 2000, 1, 2000)
    anchor3 = anchor.astype(jnp.int32).reshape(N_NODES // 2000, 1, 2000)
    W_f2p = jnp.pad(W_f2, ((0, 0), (0, NODE_DIM - 1)))
    b_f2p = jnp.pad(b_f2, (0, NODE_DIM - 1)).reshape(1, NODE_DIM)
    scores_m, x1, x2 = _readout_tc(h, batch3, anchor3, W_f1,
                                   b_f1.reshape(1, NODE_DIM), W_f2p, b_f2p)
    return (scores_m[:, 0], h, x1, x2)


# speedup vs baseline: 7.0575x; 1.0315x over previous
import functools
import jax
import jax.numpy as jnp
from jax import lax
from jax.experimental import pallas as pl
from jax.experimental.pallas import tpu as pltpu
from jax.experimental.pallas import tpu_sc as plsc

N_NODES = 10000
N_EDGES = 320000
NODE_DIM = 128
EDGE_EMB = 16
NUM_GRAPHS = 256

NC = 2
NS = 16
NW = NC * NS
CH = 80
PER_TILE = N_EDGES // NW
N_CHUNK = PER_TILE // CH
N_PAD = 10240
ROWS_PER_TILE = N_PAD // NS
ZCH = 64

_mesh = plsc.VectorSubcoreMesh(
    core_axis_name="c", subcore_axis_name="s", num_cores=NC, num_subcores=NS)


NB = 3


@functools.partial(
    pl.kernel, mesh=_mesh,
    out_type=jax.ShapeDtypeStruct((NC * N_PAD, NODE_DIM), jnp.float32),
    scratch_types=[
        pltpu.VMEM((CH,), jnp.int32),
        pltpu.VMEM((CH,), jnp.int32),
        pltpu.VMEM((CH,), jnp.int32),
        pltpu.VMEM((CH,), jnp.int32),
        pltpu.VMEM((CH,), jnp.int32),
        pltpu.VMEM((CH,), jnp.int32),
        pltpu.VMEM((CH, NODE_DIM), jnp.float32),
        pltpu.VMEM((CH, NODE_DIM), jnp.float32),
        pltpu.VMEM((CH, NODE_DIM), jnp.float32),
        pltpu.VMEM((ZCH, NODE_DIM), jnp.float32),
        pltpu.VMEM_SHARED((N_PAD, NODE_DIM), jnp.float32),
        pltpu.SemaphoreType.DMA,
        pltpu.SemaphoreType.DMA,
        pltpu.SemaphoreType.DMA,
        pltpu.SemaphoreType.DMA,
        pltpu.SemaphoreType.DMA,
        pltpu.SemaphoreType.DMA,
    ])
def _seg_sum_sc(h_hbm, src_hbm, dst_hbm, out_hbm, s0, s1, s2, d0, d1, d2,
                r0, r1, r2, zbuf, acc_sh, g0, g1, g2, q0, q1, q2):
    c = lax.axis_index("c")
    s = lax.axis_index("s")
    wid = c * NS + s
    srcb = [s0, s1, s2]
    dstb = [d0, d1, d2]
    rowb = [r0, r1, r2]
    semb = [g0, g1, g2]
    dsem = [q0, q1, q2]
    zero16 = jnp.zeros((16,), jnp.float32)

    def zrow(i, carry):
        for j in range(8):
            zbuf[i, pl.ds(j * 16, 16)] = zero16
        return carry
    lax.fori_loop(0, ZCH, zrow, 0)

    base_r = s * ROWS_PER_TILE
    for k in range(ROWS_PER_TILE // ZCH):
        pltpu.sync_copy(zbuf, acc_sh.at[pl.ds(base_r + k * ZCH, ZCH)])
    plsc.subcore_barrier()

    ebase = wid * PER_TILE

    pltpu.sync_copy(src_hbm.at[pl.ds(ebase, CH)], s0)
    pltpu.async_copy(h_hbm.at[s0], r0, g0)
    pltpu.async_copy(dst_hbm.at[pl.ds(ebase, CH)], d0, q0)
    pltpu.sync_copy(src_hbm.at[pl.ds(ebase + CH, CH)], s1)
    pltpu.async_copy(h_hbm.at[s1], r1, g1)
    pltpu.async_copy(dst_hbm.at[pl.ds(ebase + CH, CH)], d1, q1)

    def group(g, carry):
        for b in range(NB):
            v = g * NB + b
            bp = (b + 2) % NB

            @pl.when(v + 2 < N_CHUNK)
            def _prefetch():
                pltpu.sync_copy(
                    src_hbm.at[pl.ds(ebase + (v + 2) * CH, CH)], srcb[bp])
                pltpu.async_copy(h_hbm.at[srcb[bp]], rowb[bp], semb[bp])
                pltpu.async_copy(
                    dst_hbm.at[pl.ds(ebase + (v + 2) * CH, CH)], dstb[bp],
                    dsem[bp])

            @pl.when(v < N_CHUNK)
            def _visit():
                pltpu.make_async_copy(
                    dst_hbm.at[pl.ds(ebase, CH)], dstb[b], dsem[b]).wait()
                pltpu.make_async_copy(h_hbm.at[srcb[b]], rowb[b],
                                      semb[b]).wait()
                pltpu.sync_copy(rowb[b], acc_sh.at[dstb[b]], add=True)
        return carry
    lax.fori_loop(0, (N_CHUNK + NB - 1) // NB, group, 0)
    plsc.subcore_barrier()

    out_base = c * N_PAD + base_r
    for k in range(ROWS_PER_TILE // ZCH):
        pltpu.sync_copy(acc_sh.at[pl.ds(base_r + k * ZCH, ZCH)], zbuf)
        pltpu.sync_copy(zbuf, out_hbm.at[pl.ds(out_base + k * ZCH, ZCH)])


@functools.partial(
    pl.kernel, mesh=_mesh,
    out_type=jax.ShapeDtypeStruct((NC * N_PAD, NODE_DIM), jnp.float32),
    scratch_types=[
        pltpu.VMEM((CH,), jnp.int32),
        pltpu.VMEM((CH,), jnp.int32),
        pltpu.VMEM((CH,), jnp.int32),
        pltpu.VMEM((CH, NODE_DIM), jnp.float32),
        pltpu.VMEM((CH, NODE_DIM), jnp.float32),
        pltpu.VMEM((CH, NODE_DIM), jnp.float32),
        pltpu.VMEM((ZCH, NODE_DIM), jnp.float32),
        pltpu.VMEM_SHARED((N_PAD, NODE_DIM), jnp.float32),
        pltpu.SemaphoreType.DMA,
        pltpu.SemaphoreType.DMA,
        pltpu.SemaphoreType.DMA,
        pltpu.SemaphoreType.DMA,
        pltpu.SemaphoreType.DMA,
        pltpu.SemaphoreType.DMA,
    ])
def _edge_agg_sc(ef_hbm, dst_hbm, out_hbm, d0, d1, d2, r0, r1, r2, zbuf,
                 acc_sh, g0, g1, g2, q0, q1, q2):
    c = lax.axis_index("c")
    s = lax.axis_index("s")
    wid = c * NS + s
    dstb = [d0, d1, d2]
    rowb = [r0, r1, r2]
    semb = [g0, g1, g2]
    dsem = [q0, q1, q2]
    zero16 = jnp.zeros((16,), jnp.float32)

    def zrow(i, carry):
        for j in range(8):
            zbuf[i, pl.ds(j * 16, 16)] = zero16
        return carry
    lax.fori_loop(0, ZCH, zrow, 0)

    base_r = s * ROWS_PER_TILE
    for k in range(ROWS_PER_TILE // ZCH):
        pltpu.sync_copy(zbuf, acc_sh.at[pl.ds(base_r + k * ZCH, ZCH)])
    plsc.subcore_barrier()

    ebase = wid * PER_TILE

    pltpu.async_copy(ef_hbm.at[pl.ds(ebase, CH)], r0, g0)
    pltpu.async_copy(dst_hbm.at[pl.ds(ebase, CH)], d0, q0)
    pltpu.async_copy(ef_hbm.at[pl.ds(ebase + CH, CH)], r1, g1)
    pltpu.async_copy(dst_hbm.at[pl.ds(ebase + CH, CH)], d1, q1)

    def group(g, carry):
        for b in range(NB):
            v = g * NB + b
            bp = (b + 2) % NB

            @pl.when(v + 2 < N_CHUNK)
            def _prefetch():
                pltpu.async_copy(
                    ef_hbm.at[pl.ds(ebase + (v + 2) * CH, CH)], rowb[bp],
                    semb[bp])
                pltpu.async_copy(
                    dst_hbm.at[pl.ds(ebase + (v + 2) * CH, CH)], dstb[bp],
                    dsem[bp])

            @pl.when(v < N_CHUNK)
            def _visit():
                pltpu.make_async_copy(
                    dst_hbm.at[pl.ds(ebase, CH)], dstb[b], dsem[b]).wait()
                pltpu.make_async_copy(
                    ef_hbm.at[pl.ds(ebase, CH)], rowb[b], semb[b]).wait()
                pltpu.sync_copy(rowb[b], acc_sh.at[dstb[b]], add=True)
        return carry
    lax.fori_loop(0, (N_CHUNK + NB - 1) // NB, group, 0)
    plsc.subcore_barrier()

    out_base = c * N_PAD + base_r
    for k in range(ROWS_PER_TILE // ZCH):
        pltpu.sync_copy(acc_sh.at[pl.ds(base_r + k * ZCH, ZCH)], zbuf)
        pltpu.sync_copy(zbuf, out_hbm.at[pl.ds(out_base + k * ZCH, ZCH)])


def _matmul_relu_body(x_ref, w_ref, b_ref, o_ref):
    y = jnp.dot(x_ref[...], w_ref[...], preferred_element_type=jnp.float32,
            precision=lax.Precision.HIGHEST)
    o_ref[...] = jnp.maximum(y + b_ref[...], 0.0)


def _ef_tc(ef_raw, w_pad, b_pad):
    blk = 2000
    grid = N_EDGES // blk
    return pl.pallas_call(
        _matmul_relu_body,
        grid=(grid,),
        in_specs=[
            pl.BlockSpec((blk, EDGE_EMB), lambda i: (i, 0)),
            pl.BlockSpec((EDGE_EMB, 128), lambda i: (0, 0)),
            pl.BlockSpec((1, 128), lambda i: (0, 0)),
        ],
        out_specs=pl.BlockSpec((blk, 128), lambda i: (i, 0)),
        out_shape=jax.ShapeDtypeStruct((N_EDGES, 128), jnp.float32),
    )(ef_raw, w_pad, b_pad)


def _layer_body(a_ref, e_ref, wh_ref, we_ref, b_ref, o_ref):
    agg = a_ref[0] + a_ref[1]
    ec = e_ref[0] + e_ref[1]
    efa = ec[:, 0:EDGE_EMB]
    cnt = ec[:, EDGE_EMB:EDGE_EMB + 1]
    inv = 1.0 / jnp.maximum(cnt, 1.0)
    y = (jnp.dot(agg, wh_ref[...], preferred_element_type=jnp.float32,
            precision=lax.Precision.HIGHEST)
         + jnp.dot(efa, we_ref[...], preferred_element_type=jnp.float32,
            precision=lax.Precision.HIGHEST)
         + cnt * b_ref[...])
    o_ref[...] = jnp.maximum(y * inv, 0.0)


def _layer_tc(aggp, ecp, Wh, We, b2d):
    blk = 2000
    grid = N_NODES // blk
    return pl.pallas_call(
        _layer_body,
        grid=(grid,),
        in_specs=[
            pl.BlockSpec((2, blk, NODE_DIM), lambda i: (0, i, 0)),
            pl.BlockSpec((2, blk, NODE_DIM), lambda i: (0, i, 0)),
            pl.BlockSpec((NODE_DIM, NODE_DIM), lambda i: (0, 0)),
            pl.BlockSpec((EDGE_EMB, NODE_DIM), lambda i: (0, 0)),
            pl.BlockSpec((1, NODE_DIM), lambda i: (0, 0)),
        ],
        out_specs=pl.BlockSpec((blk, NODE_DIM), lambda i: (i, 0)),
        out_shape=jax.ShapeDtypeStruct((N_NODES, NODE_DIM), jnp.float32),
    )(aggp, ecp, Wh, We, b2d)


def _readout_body(h_ref, b_ref, a_ref, wf1_ref, bf1_ref, wf2_ref, bf2_ref,
                  s1_ref, x1_ref, x2_ref, s1acc, c1acc, s2acc, c2acc):
    i = pl.program_id(0)
    blk = h_ref.shape[0]

    @pl.when(i == 0)
    def _init():
        s1acc[...] = jnp.zeros_like(s1acc)
        c1acc[...] = jnp.zeros_like(c1acc)
        s2acc[...] = jnp.zeros_like(s2acc)
        c2acc[...] = jnp.zeros_like(c2acc)

    batch = b_ref[0, 0, :]
    anchor = a_ref[0, 0, :]
    ga = lax.broadcasted_iota(jnp.int32, (NUM_GRAPHS, blk), 0)
    m_g = ga == batch[None, :]
    oh1 = jnp.where(m_g & (anchor[None, :] == 0), 1.0, 0.0)
    oh2 = jnp.where(m_g & (anchor[None, :] == 1), 1.0, 0.0)
    h = h_ref[...]
    ones = jnp.ones((blk, NODE_DIM), jnp.float32)
    s1acc[...] += jnp.dot(oh1, h, preferred_element_type=jnp.float32,
            precision=lax.Precision.HIGHEST)
    c1acc[...] += jnp.dot(oh1, ones, preferred_element_type=jnp.float32,
            precision=lax.Precision.HIGHEST)
    s2acc[...] += jnp.dot(oh2, h, preferred_element_type=jnp.float32,
            precision=lax.Precision.HIGHEST)
    c2acc[...] += jnp.dot(oh2, ones, preferred_element_type=jnp.float32,
            precision=lax.Precision.HIGHEST)

    @pl.when(i == pl.num_programs(0) - 1)
    def _final():
        x1 = s1acc[...] / jnp.maximum(c1acc[...], 1.0)
        x2 = s2acc[...] / jnp.maximum(c2acc[...], 1.0)
        xsub = x1 - x2
        t = jnp.maximum(
            jnp.dot(xsub, wf1_ref[...], preferred_element_type=jnp.float32,
            precision=lax.Precision.HIGHEST)
            + bf1_ref[...], 0.0)
        s1_ref[...] = (jnp.dot(t, wf2_ref[...],
                               preferred_element_type=jnp.float32,
            precision=lax.Precision.HIGHEST)
                       + bf2_ref[...])
        x1_ref[...] = x1
        x2_ref[...] = x2


def _readout_tc(h, batch3, anchor3, W_f1, b_f1_2d, W_f2p, b_f2p):
    blk = 2000
    grid = N_NODES // blk
    G = NUM_GRAPHS
    return pl.pallas_call(
        _readout_body,
        grid=(grid,),
        in_specs=[
            pl.BlockSpec((blk, NODE_DIM), lambda i: (i, 0)),
            pl.BlockSpec((1, 1, blk), lambda i: (i, 0, 0)),
            pl.BlockSpec((1, 1, blk), lambda i: (i, 0, 0)),
            pl.BlockSpec((NODE_DIM, NODE_DIM), lambda i: (0, 0)),
            pl.BlockSpec((1, NODE_DIM), lambda i: (0, 0)),
            pl.BlockSpec((NODE_DIM, NODE_DIM), lambda i: (0, 0)),
            pl.BlockSpec((1, NODE_DIM), lambda i: (0, 0)),
        ],
        out_specs=[
            pl.BlockSpec((G, NODE_DIM), lambda i: (0, 0)),
            pl.BlockSpec((G, NODE_DIM), lambda i: (0, 0)),
            pl.BlockSpec((G, NODE_DIM), lambda i: (0, 0)),
        ],
        out_shape=[
            jax.ShapeDtypeStruct((G, NODE_DIM), jnp.float32),
            jax.ShapeDtypeStruct((G, NODE_DIM), jnp.float32),
            jax.ShapeDtypeStruct((G, NODE_DIM), jnp.float32),
        ],
        scratch_shapes=[
            pltpu.VMEM((G, NODE_DIM), jnp.float32),
            pltpu.VMEM((G, NODE_DIM), jnp.float32),
            pltpu.VMEM((G, NODE_DIM), jnp.float32),
            pltpu.VMEM((G, NODE_DIM), jnp.float32),
        ],
    )(h, batch3, anchor3, W_f1, b_f1_2d, W_f2p, b_f2p)


def kernel(x, edge_index, edge_features, batch, anchor, num_graphs,
           W_ef, b_ef, conv_Ws, conv_bs, W_f1, b_f1, W_f2, b_f2):
    src = edge_index[0].astype(jnp.int32)
    dst = edge_index[1].astype(jnp.int32)

    w_pad = jnp.pad(W_ef, ((0, 0), (0, 128 - EDGE_EMB)))
    b_pad = jnp.concatenate(
        [b_ef, jnp.ones((EDGE_EMB,), jnp.float32),
         jnp.zeros((128 - 2 * EDGE_EMB,), jnp.float32)]).reshape(1, 128)
    ef128 = _ef_tc(edge_features, w_pad, b_pad)

    ecp = _edge_agg_sc(ef128, dst).reshape(NC, N_PAD, NODE_DIM)

    h = x
    for W, b in zip(conv_Ws, conv_bs):
        Wh = W[:-EDGE_EMB, :]
        We = W[-EDGE_EMB:, :]
        aggp = _seg_sum_sc(h, src, dst).reshape(NC, N_PAD, NODE_DIM)
        h = _layer_tc(aggp, ecp, Wh, We, b.reshape(1, NODE_DIM))

    batch3 = batch.astype(jnp.int32).reshape(N_NODES // 2000, 1, 2000)
    anchor3 = anchor.astype(jnp.int32).reshape(N_NODES // 2000, 1, 2000)
    W_f2p = jnp.pad(W_f2, ((0, 0), (0, NODE_DIM - 1)))
    b_f2p = jnp.pad(b_f2, (0, NODE_DIM - 1)).reshape(1, NODE_DIM)
    scores_m, x1, x2 = _readout_tc(h, batch3, anchor3, W_f1,
                                   b_f1.reshape(1, NODE_DIM), W_f2p, b_f2p)
    return (scores_m[:, 0], h, x1, x2)
